# Initial kernel scaffold; baseline (speedup 1.0000x reference)
import jax
import jax.numpy as jnp
from jax.experimental import pallas as pl


def kernel(x, codebook):
    raise NotImplementedError("write your pallas kernel here")


# speedup vs baseline: 1.1752x; 1.1752x over previous
import functools

import jax
import jax.numpy as jnp
from jax.experimental import pallas as pl
from jax.experimental.pallas import tpu as pltpu

_DIM = 256
_K = 1024
_TOK = 1024
_NSTEP = 16
_NTOTAL = _NSTEP * _TOK
_NUMEL = _NTOTAL * _DIM


def _vq_body(x_ref, cb_ref, q_ref, idx_ref, loss_ref, ent_ref,
             counts_ref, sse_ref):
    s = pl.program_id(0)

    @pl.when(s == 0)
    def _init():
        counts_ref[...] = jnp.zeros_like(counts_ref)
        sse_ref[0] = jnp.float32(0.0)

    X = x_ref[0]
    C = cb_ref[...]

    mm = jax.lax.dot_general(C, X, (((1,), (0,)), ((), ())),
                             preferred_element_type=jnp.float32)
    rn = jnp.sum(X * X, axis=0)
    cn = jnp.sum(C * C, axis=1)
    d = (rn[None, :] - 2.0 * mm) + cn[:, None]

    dmin = jnp.min(d, axis=0)
    row = jax.lax.broadcasted_iota(jnp.int32, (_K, _TOK), 0)
    idx = jnp.min(jnp.where(d == dmin[None, :], row, _K), axis=0)
    idx_ref[0, 0] = idx

    O = (row == idx[None, :]).astype(jnp.float32)
    Q = jax.lax.dot_general(C, O, (((0,), (0,)), ((), ())),
                            preferred_element_type=jnp.float32,
                            precision=jax.lax.Precision.HIGHEST)
    q_ref[0] = X + (Q - X)

    counts_ref[...] += jnp.sum(O, axis=1)[None, :]
    sse_ref[0] += jnp.sum(dmin)

    @pl.when(s == _NSTEP - 1)
    def _fin():
        loss_ref[...] = jnp.full((1, 1), sse_ref[0] * jnp.float32(1.25 / _NUMEL),
                                 jnp.float32)
        counts = counts_ref[0, :]
        total = jnp.maximum(jnp.sum(counts), 1.0)
        probs = counts / total
        safe = jnp.maximum(probs, 1e-30)
        ent = -jnp.sum(jnp.where(probs > 0,
                                 probs * (jnp.log(safe) / jnp.log(2.0)),
                                 0.0))
        ent_ref[...] = jnp.full((1, 1), ent, jnp.float32)


@jax.jit
def kernel(x, codebook):
    xr = x.reshape(_NSTEP, _DIM, _TOK)
    q, idx, loss, ent = pl.pallas_call(
        _vq_body,
        grid=(_NSTEP,),
        in_specs=[
            pl.BlockSpec((1, _DIM, _TOK), lambda s: (s, 0, 0)),
            pl.BlockSpec((_K, _DIM), lambda s: (0, 0)),
        ],
        out_specs=[
            pl.BlockSpec((1, _DIM, _TOK), lambda s: (s, 0, 0)),
            pl.BlockSpec((1, 1, _TOK), lambda s: (s, 0, 0)),
            pl.BlockSpec((1, 1), lambda s: (0, 0)),
            pl.BlockSpec((1, 1), lambda s: (0, 0)),
        ],
        out_shape=[
            jax.ShapeDtypeStruct((_NSTEP, _DIM, _TOK), jnp.float32),
            jax.ShapeDtypeStruct((_NSTEP, 1, _TOK), jnp.int32),
            jax.ShapeDtypeStruct((1, 1), jnp.float32),
            jax.ShapeDtypeStruct((1, 1), jnp.float32),
        ],
        scratch_shapes=[
            pltpu.VMEM((1, _K), jnp.float32),
            pltpu.SMEM((1,), jnp.float32),
        ],
    )(xr, codebook)
    quantized_st = q.reshape(x.shape)
    indices = idx.reshape(_NTOTAL)
    return quantized_st, indices, loss[0, 0], ent[0, 0]


# speedup vs baseline: 1.8659x; 1.5877x over previous
import functools

import jax
import jax.numpy as jnp
from jax.experimental import pallas as pl
from jax.experimental.pallas import tpu as pltpu

_DIM = 256
_K = 1024
_TOK = 1024
_NSTEP = 16
_NTOTAL = _NSTEP * _TOK
_NUMEL = _NTOTAL * _DIM


def _vq_body(x_ref, cb_ref, q_ref, idx_ref, loss_ref, ent_ref,
             counts_ref, sse_ref):
    s = pl.program_id(0)

    @pl.when(s == 0)
    def _init():
        counts_ref[...] = jnp.zeros_like(counts_ref)
        sse_ref[0] = jnp.float32(0.0)

    X = x_ref[0]
    C = cb_ref[...]

    mm = jax.lax.dot_general(C, X, (((1,), (0,)), ((), ())),
                             preferred_element_type=jnp.float32)
    rn = jnp.sum(X * X, axis=0)
    cn = jnp.sum(C * C, axis=1)
    d = (rn[None, :] - 2.0 * mm) + cn[:, None]

    dmin = jnp.min(d, axis=0)
    row = jax.lax.broadcasted_iota(jnp.int32, (_K, _TOK), 0)
    idx = jnp.min(jnp.where(d == dmin[None, :], row, _K), axis=0)
    idx_ref[0, 0] = idx

    O = (row == idx[None, :]).astype(jnp.float32)
    Q = jax.lax.dot_general(C, O, (((0,), (0,)), ((), ())),
                            preferred_element_type=jnp.float32)
    q_ref[0] = X + (Q - X)

    ones_t = jnp.ones((_TOK, 1), jnp.float32)
    counts_ref[...] += jax.lax.dot_general(O, ones_t, (((1,), (0,)), ((), ())),
                                           preferred_element_type=jnp.float32)
    sse_ref[0] += jnp.sum(dmin)

    @pl.when(s == _NSTEP - 1)
    def _fin():
        loss_ref[...] = jnp.full((1, 1), sse_ref[0] * jnp.float32(1.25 / _NUMEL),
                                 jnp.float32)
        counts = counts_ref[:, 0]
        total = jnp.maximum(jnp.sum(counts), 1.0)
        probs = counts / total
        safe = jnp.maximum(probs, 1e-30)
        ent = -jnp.sum(jnp.where(probs > 0,
                                 probs * (jnp.log(safe) / jnp.log(2.0)),
                                 0.0))
        ent_ref[...] = jnp.full((1, 1), ent, jnp.float32)


@jax.jit
def kernel(x, codebook):
    xr = x.reshape(_NSTEP, _DIM, _TOK)
    q, idx, loss, ent = pl.pallas_call(
        _vq_body,
        grid=(_NSTEP,),
        in_specs=[
            pl.BlockSpec((1, _DIM, _TOK), lambda s: (s, 0, 0)),
            pl.BlockSpec((_K, _DIM), lambda s: (0, 0)),
        ],
        out_specs=[
            pl.BlockSpec((1, _DIM, _TOK), lambda s: (s, 0, 0)),
            pl.BlockSpec((1, 1, _TOK), lambda s: (s, 0, 0)),
            pl.BlockSpec((1, 1), lambda s: (0, 0)),
            pl.BlockSpec((1, 1), lambda s: (0, 0)),
        ],
        out_shape=[
            jax.ShapeDtypeStruct((_NSTEP, _DIM, _TOK), jnp.float32),
            jax.ShapeDtypeStruct((_NSTEP, 1, _TOK), jnp.int32),
            jax.ShapeDtypeStruct((1, 1), jnp.float32),
            jax.ShapeDtypeStruct((1, 1), jnp.float32),
        ],
        scratch_shapes=[
            pltpu.VMEM((_K, 1), jnp.float32),
            pltpu.SMEM((1,), jnp.float32),
        ],
    )(xr, codebook)
    quantized_st = q.reshape(x.shape)
    indices = idx.reshape(_NTOTAL)
    return quantized_st, indices, loss[0, 0], ent[0, 0]


# speedup vs baseline: 1.9234x; 1.0308x over previous
import functools

import jax
import jax.numpy as jnp
from jax.experimental import pallas as pl
from jax.experimental.pallas import tpu as pltpu

_DIM = 256
_K = 1024
_TOK = 1024
_UNROLL = 2
_NSTEP = 16 // _UNROLL
_NTOTAL = 16 * _TOK
_NUMEL = _NTOTAL * _DIM


def _vq_body(x_ref, cb_ref, q_ref, idx_ref, loss_ref, ent_ref,
             counts_ref, sse_ref):
    s = pl.program_id(0)

    @pl.when(s == 0)
    def _init():
        counts_ref[...] = jnp.zeros_like(counts_ref)
        sse_ref[0] = jnp.float32(0.0)

    C = cb_ref[...]
    cn = jnp.sum(C * C, axis=1)
    row = jax.lax.broadcasted_iota(jnp.int32, (_K, _TOK), 0)
    ones_t = jnp.ones((_TOK, 1), jnp.float32)

    for u in range(_UNROLL):
        X = x_ref[u]

        mm = jax.lax.dot_general(C, X, (((1,), (0,)), ((), ())),
                                 preferred_element_type=jnp.float32)
        rn = jnp.sum(X * X, axis=0)
        d = (rn[None, :] - 2.0 * mm) + cn[:, None]

        dmin = jnp.min(d, axis=0)
        idx = jnp.min(jnp.where(d == dmin[None, :], row, _K), axis=0)
        idx_ref[u, 0] = idx

        O = (row == idx[None, :]).astype(jnp.float32)
        Q = jax.lax.dot_general(C, O, (((0,), (0,)), ((), ())),
                                preferred_element_type=jnp.float32)
        q_ref[u] = X + (Q - X)

        counts_ref[...] += jax.lax.dot_general(
            O, ones_t, (((1,), (0,)), ((), ())),
            preferred_element_type=jnp.float32)
        sse_ref[0] += jnp.sum(dmin)

    @pl.when(s == _NSTEP - 1)
    def _fin():
        loss_ref[...] = jnp.full((1, 1), sse_ref[0] * jnp.float32(1.25 / _NUMEL),
                                 jnp.float32)
        counts = counts_ref[:, 0]
        total = jnp.maximum(jnp.sum(counts), 1.0)
        probs = counts / total
        safe = jnp.maximum(probs, 1e-30)
        ent = -jnp.sum(jnp.where(probs > 0,
                                 probs * (jnp.log(safe) / jnp.log(2.0)),
                                 0.0))
        ent_ref[...] = jnp.full((1, 1), ent, jnp.float32)


@jax.jit
def kernel(x, codebook):
    xr = x.reshape(16, _DIM, _TOK)
    q, idx, loss, ent = pl.pallas_call(
        _vq_body,
        grid=(_NSTEP,),
        in_specs=[
            pl.BlockSpec((_UNROLL, _DIM, _TOK), lambda s: (s, 0, 0)),
            pl.BlockSpec((_K, _DIM), lambda s: (0, 0)),
        ],
        out_specs=[
            pl.BlockSpec((_UNROLL, _DIM, _TOK), lambda s: (s, 0, 0)),
            pl.BlockSpec((_UNROLL, 1, _TOK), lambda s: (s, 0, 0)),
            pl.BlockSpec((1, 1), lambda s: (0, 0)),
            pl.BlockSpec((1, 1), lambda s: (0, 0)),
        ],
        out_shape=[
            jax.ShapeDtypeStruct((16, _DIM, _TOK), jnp.float32),
            jax.ShapeDtypeStruct((16, 1, _TOK), jnp.int32),
            jax.ShapeDtypeStruct((1, 1), jnp.float32),
            jax.ShapeDtypeStruct((1, 1), jnp.float32),
        ],
        scratch_shapes=[
            pltpu.VMEM((_K, 1), jnp.float32),
            pltpu.SMEM((1,), jnp.float32),
        ],
    )(xr, codebook)
    quantized_st = q.reshape(x.shape)
    indices = idx.reshape(_NTOTAL)
    return quantized_st, indices, loss[0, 0], ent[0, 0]


# speedup vs baseline: 1.9240x; 1.0003x over previous
import functools

import jax
import jax.numpy as jnp
from jax.experimental import pallas as pl
from jax.experimental.pallas import tpu as pltpu

_DIM = 256
_K = 1024
_TOK = 1024
_UNROLL = 4
_NSTEP = 16 // _UNROLL
_NTOTAL = 16 * _TOK
_NUMEL = _NTOTAL * _DIM


def _vq_body(x_ref, cb_ref, q_ref, idx_ref, loss_ref, ent_ref,
             counts_ref, sse_ref):
    s = pl.program_id(0)

    @pl.when(s == 0)
    def _init():
        counts_ref[...] = jnp.zeros_like(counts_ref)
        sse_ref[0] = jnp.float32(0.0)

    C = cb_ref[...]
    cn = jnp.sum(C * C, axis=1)
    row = jax.lax.broadcasted_iota(jnp.int32, (_K, _TOK), 0)
    ones_t = jnp.ones((_TOK, 1), jnp.float32)

    for u in range(_UNROLL):
        X = x_ref[u]

        mm = jax.lax.dot_general(C, X, (((1,), (0,)), ((), ())),
                                 preferred_element_type=jnp.float32)
        rn = jnp.sum(X * X, axis=0)
        d = (rn[None, :] - 2.0 * mm) + cn[:, None]

        dmin = jnp.min(d, axis=0)
        idx = jnp.min(jnp.where(d == dmin[None, :], row, _K), axis=0)
        idx_ref[u, 0] = idx

        O = (row == idx[None, :]).astype(jnp.float32)
        Q = jax.lax.dot_general(C, O, (((0,), (0,)), ((), ())),
                                preferred_element_type=jnp.float32)
        q_ref[u] = X + (Q - X)

        counts_ref[...] += jax.lax.dot_general(
            O, ones_t, (((1,), (0,)), ((), ())),
            preferred_element_type=jnp.float32)
        sse_ref[0] += jnp.sum(dmin)

    @pl.when(s == _NSTEP - 1)
    def _fin():
        loss_ref[...] = jnp.full((1, 1), sse_ref[0] * jnp.float32(1.25 / _NUMEL),
                                 jnp.float32)
        counts = counts_ref[:, 0]
        total = jnp.maximum(jnp.sum(counts), 1.0)
        probs = counts / total
        safe = jnp.maximum(probs, 1e-30)
        ent = -jnp.sum(jnp.where(probs > 0,
                                 probs * (jnp.log(safe) / jnp.log(2.0)),
                                 0.0))
        ent_ref[...] = jnp.full((1, 1), ent, jnp.float32)


@jax.jit
def kernel(x, codebook):
    xr = x.reshape(16, _DIM, _TOK)
    q, idx, loss, ent = pl.pallas_call(
        _vq_body,
        grid=(_NSTEP,),
        in_specs=[
            pl.BlockSpec((_UNROLL, _DIM, _TOK), lambda s: (s, 0, 0)),
            pl.BlockSpec((_K, _DIM), lambda s: (0, 0)),
        ],
        out_specs=[
            pl.BlockSpec((_UNROLL, _DIM, _TOK), lambda s: (s, 0, 0)),
            pl.BlockSpec((_UNROLL, 1, _TOK), lambda s: (s, 0, 0)),
            pl.BlockSpec((1, 1), lambda s: (0, 0)),
            pl.BlockSpec((1, 1), lambda s: (0, 0)),
        ],
        out_shape=[
            jax.ShapeDtypeStruct((16, _DIM, _TOK), jnp.float32),
            jax.ShapeDtypeStruct((16, 1, _TOK), jnp.int32),
            jax.ShapeDtypeStruct((1, 1), jnp.float32),
            jax.ShapeDtypeStruct((1, 1), jnp.float32),
        ],
        scratch_shapes=[
            pltpu.VMEM((_K, 1), jnp.float32),
            pltpu.SMEM((1,), jnp.float32),
        ],
    )(xr, codebook)
    quantized_st = q.reshape(x.shape)
    indices = idx.reshape(_NTOTAL)
    return quantized_st, indices, loss[0, 0], ent[0, 0]
